# baseline (speedup 1.0000x reference)
import functools

import jax
import jax.numpy as jnp
from jax import lax
from jax.experimental import pallas as pl
from jax.experimental.pallas import tpu as pltpu
from jax.experimental.pallas import tpu_sc as plsc

_B = 16384
_V = 100000
_D = 317
_H1, _H2, _H3 = 1024, 512, 256

_NC, _NS = 2, 16
_NW = _NC * _NS
_BPW = _B // _NW
_CHUNK = 128
_NCHUNK = _BPW // _CHUNK


def _sc_gather_main(table, genre):
  mesh = plsc.VectorSubcoreMesh(core_axis_name="c", subcore_axis_name="s")

  @functools.partial(
      pl.kernel,
      mesh=mesh,
      out_type=jax.ShapeDtypeStruct((_B, 256), jnp.float32),
      scratch_types=[
          pltpu.VMEM((_BPW,), jnp.int32),
          pltpu.VMEM((_CHUNK, 256), jnp.float32),
          pltpu.VMEM((_CHUNK, 256), jnp.float32),
          pltpu.SemaphoreType.DMA,
          pltpu.SemaphoreType.DMA,
      ],
  )
  def gather_kernel(table_hbm, idx_hbm, out_hbm, idx_v, buf0, buf1,
                    sem0, sem1):
    wid = lax.axis_index("s") * _NC + lax.axis_index("c")
    base = wid * _BPW
    pltpu.sync_copy(idx_hbm.at[pl.ds(base, _BPW)], idx_v)

    bufs = (buf0, buf1)
    sems = (sem0, sem1)

    def fire(i, buf, sem):
      idx = idx_v.at[pl.ds(i * _CHUNK, _CHUNK)]
      a = pltpu.async_copy(table_hbm.at[idx, pl.ds(0, 128)],
                           buf.at[:, pl.ds(0, 128)], sem)
      b = pltpu.async_copy(table_hbm.at[idx, pl.ds(128, 128)],
                           buf.at[:, pl.ds(128, 128)], sem)
      return (a, b)

    def drain(i, handles, buf):
      for h in handles:
        h.wait()
      pltpu.sync_copy(buf, out_hbm.at[pl.ds(base + i * _CHUNK, _CHUNK)])

    handles = [None, None]
    handles[0] = fire(0, bufs[0], sems[0])
    handles[1] = fire(1, bufs[1], sems[1])
    for i in range(_NCHUNK):
      drain(i, handles[i % 2], bufs[i % 2])
      nxt = i + 2
      if nxt < _NCHUNK:
        handles[nxt % 2] = fire(nxt, bufs[nxt % 2], sems[nxt % 2])

  return gather_kernel(table, genre)


def _sc_gather_tail(tailp, genre):
  mesh = plsc.VectorSubcoreMesh(core_axis_name="c", subcore_axis_name="s")

  @functools.partial(
      pl.kernel,
      mesh=mesh,
      out_type=jax.ShapeDtypeStruct((_B, 128), jnp.float32),
      scratch_types=[
          pltpu.VMEM((_BPW,), jnp.int32),
          pltpu.VMEM((_CHUNK, 128), jnp.float32),
          pltpu.VMEM((_CHUNK, 128), jnp.float32),
          pltpu.SemaphoreType.DMA,
          pltpu.SemaphoreType.DMA,
      ],
  )
  def gather_kernel(tail_hbm, idx_hbm, out_hbm, idx_v, buf0, buf1,
                    sem0, sem1):
    wid = lax.axis_index("s") * _NC + lax.axis_index("c")
    base = wid * _BPW
    pltpu.sync_copy(idx_hbm.at[pl.ds(base, _BPW)], idx_v)

    bufs = (buf0, buf1)
    sems = (sem0, sem1)

    def fire(i, buf, sem):
      idx = idx_v.at[pl.ds(i * _CHUNK, _CHUNK)]
      return (pltpu.async_copy(tail_hbm.at[idx], buf, sem),)

    def drain(i, handles, buf):
      for h in handles:
        h.wait()
      pltpu.sync_copy(buf, out_hbm.at[pl.ds(base + i * _CHUNK, _CHUNK)])

    handles = [None, None]
    handles[0] = fire(0, bufs[0], sems[0])
    handles[1] = fire(1, bufs[1], sems[1])
    for i in range(_NCHUNK):
      drain(i, handles[i % 2], bufs[i % 2])
      nxt = i + 2
      if nxt < _NCHUNK:
        handles[nxt % 2] = fire(nxt, bufs[nxt % 2], sems[nxt % 2])

  return gather_kernel(tailp, genre)


_BM = 512


def _tail_body(in_ref, out_ref):
  out_ref[...] = in_ref[...]


def _make_tail(table):
  grid = (_V // 5000,)
  return pl.pallas_call(
      _tail_body,
      grid=grid,
      in_specs=[pl.BlockSpec((5000, 128), lambda i: (i, 2))],
      out_specs=pl.BlockSpec((5000, 128), lambda i: (i, 0)),
      out_shape=jax.ShapeDtypeStruct((_V, 128), jnp.float32),
  )(table)


def _mlp_body(x_ref, w1_ref, b1_ref, w2_ref, b2_ref, w3_ref, b3_ref, out_ref):
  x = x_ref[...].astype(jnp.bfloat16)
  h = jnp.dot(x, w1_ref[...], preferred_element_type=jnp.float32) + b1_ref[...]
  h = jnp.maximum(h, 0.0)
  h = jnp.dot(h.astype(jnp.bfloat16), w2_ref[...],
              preferred_element_type=jnp.float32) + b2_ref[...]
  h = jnp.maximum(h, 0.0)
  h = jnp.dot(h.astype(jnp.bfloat16), w3_ref[...],
              preferred_element_type=jnp.float32) + b3_ref[...]
  m = jnp.max(h, axis=-1, keepdims=True)
  e = jnp.exp(h - m)
  out_ref[...] = e * (1.0 / jnp.sum(e, axis=-1, keepdims=True))


def _tc_mlp(feats, w1p, b1, w2, b2, w3, b3):
  grid = (_B // _BM,)
  return pl.pallas_call(
      _mlp_body,
      grid=grid,
      in_specs=[
          pl.BlockSpec((_BM, 8), lambda i: (i, 0)),
          pl.BlockSpec((8, _H1), lambda i: (0, 0)),
          pl.BlockSpec((1, _H1), lambda i: (0, 0)),
          pl.BlockSpec((_H1, _H2), lambda i: (0, 0)),
          pl.BlockSpec((1, _H2), lambda i: (0, 0)),
          pl.BlockSpec((_H2, _H3), lambda i: (0, 0)),
          pl.BlockSpec((1, _H3), lambda i: (0, 0)),
      ],
      out_specs=pl.BlockSpec((_BM, _H3), lambda i: (i, 0)),
      out_shape=jax.ShapeDtypeStruct((_B, _H3), jnp.float32),
  )(feats, w1p, b1, w2, b2, w3, b3)


def _asm_body(m_ref, t_ref, h_ref, out_ref):
  out_ref[:, :256] = m_ref[...]
  out_ref[:, 256:_D] = t_ref[:, :_D - 256]
  out_ref[:, _D:] = h_ref[...]


def _tc_assemble(emb_main, emb_tail, h):
  grid = (_B // _BM,)
  return pl.pallas_call(
      _asm_body,
      grid=grid,
      in_specs=[
          pl.BlockSpec((_BM, 256), lambda i: (i, 0)),
          pl.BlockSpec((_BM, 128), lambda i: (i, 0)),
          pl.BlockSpec((_BM, _H3), lambda i: (i, 0)),
      ],
      out_specs=pl.BlockSpec((_BM, _D + _H3), lambda i: (i, 0)),
      out_shape=jax.ShapeDtypeStruct((_B, _D + _H3), jnp.float32),
  )(emb_main, emb_tail, h)


def kernel(anime_id, genre, type, episodes, general_rating, members, user_id,
           user_rating, table, W1, b1, W2, b2, W3, b3):
  emb_main = _sc_gather_main(table, genre)
  tailp = _make_tail(table)
  emb_tail = _sc_gather_tail(tailp, genre)
  feats = jnp.stack(
      [anime_id, type, episodes, general_rating, members, user_id, user_rating,
       jnp.zeros_like(anime_id)], axis=-1)
  w1p = jnp.concatenate([W1, jnp.zeros((1, _H1), jnp.float32)],
                        axis=0).astype(jnp.bfloat16)
  h = _tc_mlp(feats,
              w1p, b1.reshape(1, _H1),
              W2.astype(jnp.bfloat16), b2.reshape(1, _H2),
              W3.astype(jnp.bfloat16), b3.reshape(1, _H3))
  return _tc_assemble(emb_main, emb_tail, h)


# speedup vs baseline: 1.0604x; 1.0604x over previous
import functools

import jax
import jax.numpy as jnp
from jax import lax
from jax.experimental import pallas as pl
from jax.experimental.pallas import tpu as pltpu
from jax.experimental.pallas import tpu_sc as plsc

_B = 16384
_V = 100000
_D = 317
_DP = 384
_H1, _H2, _H3 = 1024, 512, 256

_NC, _NS = 2, 16
_NW = _NC * _NS
_BPW = _B // _NW
_CHUNK = 128
_NCHUNK = _BPW // _CHUNK


def _sc_gather(table, tailp, genre):
  mesh = plsc.VectorSubcoreMesh(core_axis_name="c", subcore_axis_name="s")

  @functools.partial(
      pl.kernel,
      mesh=mesh,
      out_type=jax.ShapeDtypeStruct((_B, _DP), jnp.float32),
      scratch_types=[
          pltpu.VMEM((_BPW,), jnp.int32),
          pltpu.VMEM((_CHUNK, _DP), jnp.float32),
          pltpu.VMEM((_CHUNK, _DP), jnp.float32),
          pltpu.SemaphoreType.DMA,
          pltpu.SemaphoreType.DMA,
      ],
  )
  def gather_kernel(table_hbm, tail_hbm, idx_hbm, out_hbm, idx_v,
                    buf0, buf1, sem0, sem1):
    wid = lax.axis_index("s") * _NC + lax.axis_index("c")
    base = wid * _BPW
    pltpu.sync_copy(idx_hbm.at[pl.ds(base, _BPW)], idx_v)

    bufs = (buf0, buf1)
    sems = (sem0, sem1)

    def fire(i, buf, sem):
      idx = idx_v.at[pl.ds(i * _CHUNK, _CHUNK)]
      a = pltpu.async_copy(table_hbm.at[idx, pl.ds(0, 128)],
                           buf.at[:, pl.ds(0, 128)], sem)
      b = pltpu.async_copy(table_hbm.at[idx, pl.ds(128, 128)],
                           buf.at[:, pl.ds(128, 128)], sem)
      c = pltpu.async_copy(tail_hbm.at[idx],
                           buf.at[:, pl.ds(256, 128)], sem)
      return (a, b, c)

    def drain(i, handles, buf):
      for h in handles:
        h.wait()
      pltpu.sync_copy(buf, out_hbm.at[pl.ds(base + i * _CHUNK, _CHUNK)])

    handles = [None, None]
    handles[0] = fire(0, bufs[0], sems[0])
    handles[1] = fire(1, bufs[1], sems[1])
    for i in range(_NCHUNK):
      drain(i, handles[i % 2], bufs[i % 2])
      nxt = i + 2
      if nxt < _NCHUNK:
        handles[nxt % 2] = fire(nxt, bufs[nxt % 2], sems[nxt % 2])

  return gather_kernel(table, tailp, genre)


_BM = 1024


def _tail_body(in_ref, out_ref):
  out_ref[...] = in_ref[...]


def _make_tail(table):
  grid = (_V // 10000,)
  return pl.pallas_call(
      _tail_body,
      grid=grid,
      in_specs=[pl.BlockSpec((10000, 128), lambda i: (i, 2))],
      out_specs=pl.BlockSpec((10000, 128), lambda i: (i, 0)),
      out_shape=jax.ShapeDtypeStruct((_V, 128), jnp.float32),
  )(table)


def _mlp_body(emb_ref, x_ref, w1_ref, b1_ref, w2_ref, b2_ref, w3_ref, b3_ref,
              out_ref):
  x = x_ref[...].astype(jnp.bfloat16)
  h = jnp.dot(x, w1_ref[...], preferred_element_type=jnp.float32) + b1_ref[...]
  h = jnp.maximum(h, 0.0)
  h = jnp.dot(h.astype(jnp.bfloat16), w2_ref[...],
              preferred_element_type=jnp.float32) + b2_ref[...]
  h = jnp.maximum(h, 0.0)
  h = jnp.dot(h.astype(jnp.bfloat16), w3_ref[...],
              preferred_element_type=jnp.float32) + b3_ref[...]
  m = jnp.max(h, axis=-1, keepdims=True)
  e = jnp.exp(h - m)
  p = e * (1.0 / jnp.sum(e, axis=-1, keepdims=True))
  out_ref[:, :_D] = emb_ref[:, :_D]
  out_ref[:, _D:] = p


def _tc_mlp(emb, feats, w1p, b1, w2, b2, w3, b3):
  grid = (_B // _BM,)
  return pl.pallas_call(
      _mlp_body,
      grid=grid,
      in_specs=[
          pl.BlockSpec((_BM, _DP), lambda i: (i, 0)),
          pl.BlockSpec((_BM, 8), lambda i: (i, 0)),
          pl.BlockSpec((8, _H1), lambda i: (0, 0)),
          pl.BlockSpec((1, _H1), lambda i: (0, 0)),
          pl.BlockSpec((_H1, _H2), lambda i: (0, 0)),
          pl.BlockSpec((1, _H2), lambda i: (0, 0)),
          pl.BlockSpec((_H2, _H3), lambda i: (0, 0)),
          pl.BlockSpec((1, _H3), lambda i: (0, 0)),
      ],
      out_specs=pl.BlockSpec((_BM, _D + _H3), lambda i: (i, 0)),
      out_shape=jax.ShapeDtypeStruct((_B, _D + _H3), jnp.float32),
  )(emb, feats, w1p, b1, w2, b2, w3, b3)


def kernel(anime_id, genre, type, episodes, general_rating, members, user_id,
           user_rating, table, W1, b1, W2, b2, W3, b3):
  tailp = _make_tail(table)
  emb = _sc_gather(table, tailp, genre)
  feats = jnp.stack(
      [anime_id, type, episodes, general_rating, members, user_id, user_rating,
       jnp.zeros_like(anime_id)], axis=-1)
  w1p = jnp.concatenate([W1, jnp.zeros((1, _H1), jnp.float32)],
                        axis=0).astype(jnp.bfloat16)
  return _tc_mlp(emb, feats,
                 w1p, b1.reshape(1, _H1),
                 W2.astype(jnp.bfloat16), b2.reshape(1, _H2),
                 W3.astype(jnp.bfloat16), b3.reshape(1, _H3))
